# baseline (speedup 1.0000x reference)
import functools

import jax
import jax.numpy as jnp
from jax import lax
from jax.experimental import pallas as pl
from jax.experimental.pallas import tpu as pltpu
from jax.experimental.pallas import tpu_sc as plsc

B = 16384
K = 32
L = 16
W = 128
RPW = W // K

_info = plsc.get_sparse_core_info()
NC = _info.num_cores
NS = _info.num_subcores
NW = NC * NS
BPW = B // NW
CH = 128
NR = BPW // CH
GPR = CH // L

_mesh = plsc.VectorSubcoreMesh(core_axis_name="c", subcore_axis_name="s")


@functools.partial(
    pl.kernel,
    mesh=_mesh,
    compiler_params=pltpu.CompilerParams(
        needs_layout_passes=False,
        use_tc_tiling_on_sc=False,
    ),
    out_type=jax.ShapeDtypeStruct((B,), jnp.float32),
    scratch_types=[
        pltpu.VMEM((BPW,), jnp.int32),
        pltpu.VMEM((BPW,), jnp.int32),
        pltpu.VMEM((BPW,), jnp.int32),
        pltpu.VMEM((BPW,), jnp.int32),
        pltpu.VMEM((BPW,), jnp.int32),
        pltpu.VMEM((BPW,), jnp.int32),
        pltpu.VMEM((CH, W), jnp.float32),
        pltpu.VMEM((CH, W), jnp.float32),
        pltpu.VMEM((CH, W), jnp.float32),
        pltpu.VMEM((CH, W), jnp.float32),
        pltpu.VMEM((L,), jnp.float32),
        pltpu.VMEM((BPW,), jnp.float32),
        pltpu.SemaphoreType.DMA,
    ],
)
def _sc_predict(u_hbm, i_hbm, user_hbm, bu_hbm, movie_hbm, bm_hbm, mean_hbm,
                out_hbm, idx_u, idx_i, hu, hi, hbu, hbi, urows, mrows, burows,
                bmrows, mean_v, out_v, sem):
    wid = lax.axis_index("s") * NC + lax.axis_index("c")
    base = wid * BPW

    pltpu.sync_copy(u_hbm.at[pl.ds(base, BPW)], idx_u)
    pltpu.sync_copy(i_hbm.at[pl.ds(base, BPW)], idx_i)
    pltpu.sync_copy(mean_hbm, mean_v)

    def hi_body(c, _):
        s = pl.ds(pl.multiple_of(c * L, L), L)
        uvec = idx_u[s]
        ivec = idx_i[s]
        hu[s] = lax.shift_right_logical(uvec, 2)
        hi[s] = lax.shift_right_logical(ivec, 2)
        hbu[s] = lax.shift_right_logical(uvec, 7)
        hbi[s] = lax.shift_right_logical(ivec, 7)
        return 0

    lax.fori_loop(0, BPW // L, hi_body, 0)

    mean = mean_v[...]
    lo_col = jnp.full((L,), RPW - 1, jnp.int32)
    lo_bias = jnp.full((L,), W - 1, jnp.int32)
    rid_g = lax.iota(jnp.int32, L)

    for r in range(NR):
        s = pl.ds(r * CH, CH)
        c1 = pltpu.async_copy(user_hbm.at[hu.at[s]], urows, sem)
        c2 = pltpu.async_copy(movie_hbm.at[hi.at[s]], mrows, sem)
        c3 = pltpu.async_copy(bu_hbm.at[hbu.at[s]], burows, sem)
        c4 = pltpu.async_copy(bm_hbm.at[hbi.at[s]], bmrows, sem)
        c1.wait()
        c2.wait()
        c3.wait()
        c4.wait()

        def round_body(g, _, r=r):
            sg = pl.ds(pl.multiple_of(r * CH, L) + g * L, L)
            rid = g * L + rid_g
            uvec = idx_u[sg]
            ivec = idx_i[sg]
            ucol = (uvec & lo_col) * K
            mcol = (ivec & lo_col) * K
            acc = jnp.zeros((L,), jnp.float32)
            for k in range(K):
                uc = plsc.load_gather(urows, [rid, ucol + k])
                mc = plsc.load_gather(mrows, [rid, mcol + k])
                acc = acc + uc * mc
            bu = plsc.load_gather(burows, [rid, uvec & lo_bias])
            bm = plsc.load_gather(bmrows, [rid, ivec & lo_bias])
            out_v[sg] = acc + bu + bm + mean
            return 0

        lax.fori_loop(0, GPR, round_body, 0)

    pltpu.sync_copy(out_v, out_hbm.at[pl.ds(base, BPW)])


def _pad_rows(n):
    rows = -(-n // W)
    rows += (-rows) % 8
    return rows


def _as_rows128(x, one):
    n = x.shape[0]
    if (n * x.shape[1]) % W == 0 and x.shape[1] == K:
        return x.reshape(-1, W)
    rows = _pad_rows(n)
    return jnp.pad(x.reshape(-1) * one, (0, rows * W - n)).reshape(rows, W)


def kernel(u, i, user, bias_user, movie, bias_movie, mean):
    one = (mean - mean) + jnp.float32(1.0)
    user128 = user.reshape(-1, W)
    movie128 = movie.reshape(-1, W)
    bu128 = _as_rows128(bias_user, one)
    bm128 = _as_rows128(bias_movie, one)
    mean_v = jnp.full((L,), mean, dtype=jnp.float32)
    return _sc_predict(u, i, user128, bu128, movie128, bm128, mean_v)


# speedup vs baseline: 1.0050x; 1.0050x over previous
import functools

import jax
import jax.numpy as jnp
from jax import lax
from jax.experimental import pallas as pl
from jax.experimental.pallas import tpu as pltpu
from jax.experimental.pallas import tpu_sc as plsc

B = 16384
K = 32
L = 16
W = 128
RPW = W // K

_info = plsc.get_sparse_core_info()
NC = _info.num_cores
NS = _info.num_subcores
NW = NC * NS
BPW = B // NW
CH = 128
NR = BPW // CH
GPR = CH // L

_mesh = plsc.VectorSubcoreMesh(core_axis_name="c", subcore_axis_name="s")


@functools.partial(
    pl.kernel,
    mesh=_mesh,
    compiler_params=pltpu.CompilerParams(
        needs_layout_passes=False,
        use_tc_tiling_on_sc=False,
    ),
    out_type=jax.ShapeDtypeStruct((B,), jnp.float32),
    scratch_types=[
        pltpu.VMEM((BPW,), jnp.int32),
        pltpu.VMEM((BPW,), jnp.int32),
        pltpu.VMEM((BPW,), jnp.int32),
        pltpu.VMEM((BPW,), jnp.int32),
        pltpu.VMEM((BPW,), jnp.int32),
        pltpu.VMEM((BPW,), jnp.int32),
        pltpu.VMEM((CH, W), jnp.float32),
        pltpu.VMEM((CH, W), jnp.float32),
        pltpu.VMEM((CH, W), jnp.float32),
        pltpu.VMEM((CH, W), jnp.float32),
        pltpu.VMEM((L,), jnp.float32),
        pltpu.VMEM((BPW,), jnp.float32),
        pltpu.SemaphoreType.DMA,
    ],
)
def _sc_predict(u_hbm, i_hbm, user_hbm, bu_hbm, movie_hbm, bm_hbm, mean_hbm,
                out_hbm, idx_u, idx_i, hu, hi, hbu, hbi, urows, mrows, burows,
                bmrows, mean_v, out_v, sem):
    wid = lax.axis_index("s") * NC + lax.axis_index("c")
    base = wid * BPW

    pltpu.sync_copy(u_hbm.at[pl.ds(base, BPW)], idx_u)
    pltpu.sync_copy(i_hbm.at[pl.ds(base, BPW)], idx_i)
    pltpu.sync_copy(mean_hbm, mean_v)

    def hi_body(c, _):
        s = pl.ds(pl.multiple_of(c * L, L), L)
        uvec = idx_u[s]
        ivec = idx_i[s]
        hu[s] = uvec
        hi[s] = ivec
        hbu[s] = lax.shift_right_logical(uvec, 7)
        hbi[s] = lax.shift_right_logical(ivec, 7)
        return 0

    lax.fori_loop(0, BPW // L, hi_body, 0)

    mean = mean_v[...]
    lo_col = jnp.full((L,), RPW - 1, jnp.int32)
    lo_bias = jnp.full((L,), W - 1, jnp.int32)
    rid_g = lax.iota(jnp.int32, L)

    for r in range(NR):
        s = pl.ds(r * CH, CH)
        c1 = pltpu.async_copy(user_hbm.at[hu.at[s]], urows, sem)
        c2 = pltpu.async_copy(movie_hbm.at[hi.at[s]], mrows, sem)
        c3 = pltpu.async_copy(bu_hbm.at[hbu.at[s]], burows, sem)
        c4 = pltpu.async_copy(bm_hbm.at[hbi.at[s]], bmrows, sem)
        c1.wait()
        c2.wait()
        c3.wait()
        c4.wait()

        def round_body(g, _, r=r):
            sg = pl.ds(pl.multiple_of(r * CH, L) + g * L, L)
            rid = g * L + rid_g
            uvec = idx_u[sg]
            ivec = idx_i[sg]
            acc = jnp.zeros((L,), jnp.float32)
            for k in range(K):
                kk = jnp.full((L,), k, jnp.int32)
                uc = plsc.load_gather(urows, [rid, kk])
                mc = plsc.load_gather(mrows, [rid, kk])
                acc = acc + uc * mc
            bu = plsc.load_gather(burows, [rid, uvec & lo_bias])
            bm = plsc.load_gather(bmrows, [rid, ivec & lo_bias])
            out_v[sg] = acc + bu + bm + mean
            return 0

        lax.fori_loop(0, GPR, round_body, 0)

    pltpu.sync_copy(out_v, out_hbm.at[pl.ds(base, BPW)])


def _pad_rows(n):
    rows = -(-n // W)
    rows += (-rows) % 8
    return rows


def _as_rows128(x, one):
    n = x.shape[0]
    if (n * x.shape[1]) % W == 0 and x.shape[1] == K:
        return x.reshape(-1, W)
    rows = _pad_rows(n)
    return jnp.pad(x.reshape(-1) * one, (0, rows * W - n)).reshape(rows, W)


def kernel(u, i, user, bias_user, movie, bias_movie, mean):
    one = (mean - mean) + jnp.float32(1.0)
    user128 = jnp.pad(user, ((0, 0), (0, W - K)))
    movie128 = jnp.pad(movie, ((0, 0), (0, W - K)))
    bu128 = _as_rows128(bias_user, one)
    bm128 = _as_rows128(bias_movie, one)
    mean_v = jnp.full((L,), mean, dtype=jnp.float32)
    return _sc_predict(u, i, user128, bu128, movie128, bm128, mean_v)
